# baseline (speedup 1.0000x reference)
import jax
import jax.numpy as jnp
from jax import lax
from jax.experimental import pallas as pl
from jax.experimental.pallas import tpu as pltpu

B, S, K = 4, 2048, 768
N_CLASSES = 42
TS = 512
NSS = S // TS
NTOT = B * NSS
RG = TS // 8


def _strip_src(x_ref, idx):
    bidx = idx // NSS
    sidx = lax.rem(idx, NSS)
    return x_ref.at[pl.ds(bidx, 1), pl.ds(sidx * TS, TS), :]


def _tc_body(x_ref, m1r_ref, m2r_ref, m1_ref, m2_ref, w_ref, b_ref,
             o_ref, xb0, xb1, acc_ref, pool_ref, sem0, sem1):
    neg = jnp.float32(-jnp.inf)

    pltpu.make_async_copy(_strip_src(x_ref, 0), xb0, sem0).start()
    pltpu.make_async_copy(_strip_src(x_ref, 1), xb1, sem1).start()

    def compute(idx, buf):
        bidx = idx // NSS
        sidx = lax.rem(idx, NSS)
        x = buf[0].reshape(RG, 8, K)
        m1c = m1r_ref[pl.ds(bidx, 1), pl.ds(sidx * TS, TS), :].reshape(RG, 8, 1)
        m2c = m2r_ref[pl.ds(bidx, 1), pl.ds(sidx * TS, TS), :].reshape(RG, 8, 1)
        e1 = jnp.max(jnp.where(m1c > 0, x, neg), axis=0)
        e2 = jnp.max(jnp.where(m2c > 0, x, neg), axis=0)
        both = jnp.concatenate([e1, e2], axis=-1)

        @pl.when(sidx == 0)
        def _():
            acc_ref[...] = both

        @pl.when(sidx > 0)
        def _():
            acc_ref[...] = jnp.maximum(acc_ref[...], both)

        @pl.when(sidx == NSS - 1)
        def _():
            pool_ref[pl.ds(bidx, 1), :] = jnp.max(acc_ref[...], axis=0,
                                                  keepdims=True)

    def pair_body(g, carry):
        i0 = 2 * g
        pltpu.make_async_copy(_strip_src(x_ref, 0), xb0, sem0).wait()
        compute(i0, xb0)
        pltpu.make_async_copy(_strip_src(x_ref, jnp.minimum(i0 + 2, NTOT - 2)),
                              xb0, sem0).start()
        pltpu.make_async_copy(_strip_src(x_ref, 1), xb1, sem1).wait()
        compute(i0 + 1, xb1)
        pltpu.make_async_copy(_strip_src(x_ref, jnp.minimum(i0 + 3, NTOT - 1)),
                              xb1, sem1).start()
        return carry

    lax.fori_loop(0, NTOT // 2, pair_body, 0)

    pltpu.make_async_copy(_strip_src(x_ref, 0), xb0, sem0).wait()
    pltpu.make_async_copy(_strip_src(x_ref, 1), xb1, sem1).wait()

    c1 = jnp.sum(m1_ref[...], axis=1, keepdims=True)
    c2 = jnp.sum(m2_ref[...], axis=1, keepdims=True)
    pad1 = c1 < jnp.max(c1)
    pad2 = c2 < jnp.max(c2)
    p1 = pool_ref[:, 0:K]
    p2 = pool_ref[:, K:2 * K]
    p1 = jnp.where(pad1, jnp.maximum(p1, 0.0), p1)
    p2 = jnp.where(pad2, jnp.maximum(p2, 0.0), p2)
    dense = jnp.concatenate([p1, p2], axis=-1)
    logits = jnp.dot(dense, w_ref[...],
                     preferred_element_type=jnp.float32) + b_ref[...]
    logits = logits - jnp.max(logits, axis=-1, keepdims=True)
    e = jnp.exp(logits)
    o_ref[...] = e / jnp.sum(e, axis=-1, keepdims=True)


def kernel(bert_output, e1_mask, e2_mask, W, b):
    m1i = e1_mask.astype(jnp.int32)
    m2i = e2_mask.astype(jnp.int32)
    m1r = m1i.reshape(B, S, 1)
    m2r = m2i.reshape(B, S, 1)
    f32 = jnp.float32
    return pl.pallas_call(
        _tc_body,
        in_specs=[
            pl.BlockSpec(memory_space=pl.ANY),
            pl.BlockSpec((B, S, 1), lambda: (0, 0, 0)),
            pl.BlockSpec((B, S, 1), lambda: (0, 0, 0)),
            pl.BlockSpec((B, S), lambda: (0, 0)),
            pl.BlockSpec((B, S), lambda: (0, 0)),
            pl.BlockSpec((2 * K, N_CLASSES), lambda: (0, 0)),
            pl.BlockSpec((1, N_CLASSES), lambda: (0, 0)),
        ],
        out_specs=pl.BlockSpec((B, N_CLASSES), lambda: (0, 0)),
        out_shape=jax.ShapeDtypeStruct((B, N_CLASSES), f32),
        scratch_shapes=[
            pltpu.VMEM((1, TS, K), f32),
            pltpu.VMEM((1, TS, K), f32),
            pltpu.VMEM((8, 2 * K), f32),
            pltpu.VMEM((B, 2 * K), f32),
            pltpu.SemaphoreType.DMA,
            pltpu.SemaphoreType.DMA,
        ],
    )(bert_output, m1r, m2r, m1i, m2i, W, b.reshape(1, N_CLASSES))


# speedup vs baseline: 1.4112x; 1.4112x over previous
import jax
import jax.numpy as jnp
from jax import lax
from jax.experimental import pallas as pl
from jax.experimental.pallas import tpu as pltpu

B, S, K = 4, 2048, 768
N_CLASSES = 42
RG = S // 8


def _tc_body(x_ref, m1r_ref, m2r_ref, m1_ref, m2_ref, w_ref, b_ref,
             o_ref, pool_ref):
    bi = pl.program_id(0)
    neg = jnp.float32(-jnp.inf)

    x = x_ref[0].reshape(RG, 8, K)
    m1c = m1r_ref[0].reshape(RG, 8, 1)
    m2c = m2r_ref[0].reshape(RG, 8, 1)
    e1 = jnp.max(jnp.where(m1c > 0, x, neg), axis=0)
    e2 = jnp.max(jnp.where(m2c > 0, x, neg), axis=0)
    row = pl.ds(bi, 1)
    pool_ref[row, 0:K] = jnp.max(e1, axis=0, keepdims=True)
    pool_ref[row, K:2 * K] = jnp.max(e2, axis=0, keepdims=True)

    @pl.when(bi == B - 1)
    def _():
        c1 = jnp.sum(m1_ref[...], axis=1, keepdims=True)
        c2 = jnp.sum(m2_ref[...], axis=1, keepdims=True)
        pad1 = c1 < jnp.max(c1)
        pad2 = c2 < jnp.max(c2)
        p1 = pool_ref[:, 0:K]
        p2 = pool_ref[:, K:2 * K]
        p1 = jnp.where(pad1, jnp.maximum(p1, 0.0), p1)
        p2 = jnp.where(pad2, jnp.maximum(p2, 0.0), p2)
        dense = jnp.concatenate([p1, p2], axis=-1)
        logits = jnp.dot(dense, w_ref[...],
                         preferred_element_type=jnp.float32) + b_ref[...]
        logits = logits - jnp.max(logits, axis=-1, keepdims=True)
        e = jnp.exp(logits)
        o_ref[...] = e / jnp.sum(e, axis=-1, keepdims=True)


def kernel(bert_output, e1_mask, e2_mask, W, b):
    m1i = e1_mask.astype(jnp.int32)
    m2i = e2_mask.astype(jnp.int32)
    m1r = m1i.reshape(B, S, 1)
    m2r = m2i.reshape(B, S, 1)
    return pl.pallas_call(
        _tc_body,
        grid=(B,),
        in_specs=[
            pl.BlockSpec((1, S, K), lambda bi: (bi, 0, 0)),
            pl.BlockSpec((1, S, 1), lambda bi: (bi, 0, 0)),
            pl.BlockSpec((1, S, 1), lambda bi: (bi, 0, 0)),
            pl.BlockSpec((B, S), lambda bi: (0, 0)),
            pl.BlockSpec((B, S), lambda bi: (0, 0)),
            pl.BlockSpec((2 * K, N_CLASSES), lambda bi: (0, 0)),
            pl.BlockSpec((1, N_CLASSES), lambda bi: (0, 0)),
        ],
        out_specs=pl.BlockSpec((B, N_CLASSES), lambda bi: (0, 0)),
        out_shape=jax.ShapeDtypeStruct((B, N_CLASSES), jnp.float32),
        scratch_shapes=[pltpu.VMEM((B, 2 * K), jnp.float32)],
    )(bert_output, m1r, m2r, m1i, m2i, W, b.reshape(1, N_CLASSES))
